# Initial kernel scaffold; baseline (speedup 1.0000x reference)
import jax
import jax.numpy as jnp
from jax.experimental import pallas as pl


def kernel(x, h, edge_index, W_e1, b_e1, W_e2, b_e2, W_n1, b_n1, W_n2, b_n2, W_m1, b_m1, W_m2, b_m2, g1, beta1, g2, beta2):
    raise NotImplementedError("write your pallas kernel here")


# speedup vs baseline: 2.5004x; 2.5004x over previous
import dataclasses
import functools

import jax
import jax.numpy as jnp
from jax import lax
from jax.experimental import pallas as pl
from jax.experimental.pallas import tpu as pltpu
from jax.experimental.pallas import tpu_sc as plsc

N = 10000
E = 320000
D = 128
H = 128

W_WIN = 128
N_WORKERS = 32
WINS = 79
E_PAD = N_WORKERS * WINS * W_WIN
NWIN = E_PAD // W_WIN
D2R = E_PAD // 128
N_PAD = 10240
DUMMY = N
ROWS_PER_SUB = N_PAD // 16
HROWS = 80


def _silu(v):
    return v * jax.nn.sigmoid(v)


def _sc_compiler_params():
    cp = pltpu.CompilerParams()
    if "needs_layout_passes" in pltpu.CompilerParams.__dataclass_fields__:
        cp = dataclasses.replace(cp, needs_layout_passes=False)
    return cp


def _node_prep_body(h_ref, w1a_ref, w1b_ref, be1_ref, g1_ref, b1_ref,
                    hn_ref, a_ref, b_ref):
    h = h_ref[...]
    mu = jnp.mean(h, axis=-1, keepdims=True)
    var = jnp.mean((h - mu) ** 2, axis=-1, keepdims=True)
    hn = (h - mu) / jnp.sqrt(var + 1e-5) * g1_ref[...] + b1_ref[...]
    hn_ref[...] = hn
    a_ref[...] = jnp.dot(hn, w1a_ref[...],
                         preferred_element_type=jnp.float32) + be1_ref[...]
    b_ref[...] = jnp.dot(hn, w1b_ref[...], preferred_element_type=jnp.float32)


def _node_prep(h, w1a, w1b, be1, g1, b1):
    blk = 1000
    grid = (N // blk,)
    rowspec = pl.BlockSpec((blk, D), lambda i: (i, 0))
    wspec = pl.BlockSpec((D, H), lambda i: (0, 0))
    vspec = pl.BlockSpec((1, H), lambda i: (0, 0))
    return pl.pallas_call(
        _node_prep_body,
        grid=grid,
        in_specs=[rowspec, wspec, wspec, vspec, vspec, vspec],
        out_specs=[rowspec, rowspec, rowspec],
        out_shape=[jax.ShapeDtypeStruct((N, D), jnp.float32)] * 3,
    )(h, w1a, w1b, be1, g1, b1)


def _sc_gather(ta, tb, xp0, xp1, xp2, wd1, row2d, col2d):
    mesh = plsc.VectorSubcoreMesh(core_axis_name="core",
                                  subcore_axis_name="subcore")

    @functools.partial(
        pl.kernel,
        mesh=mesh,
        out_type=[jax.ShapeDtypeStruct((E_PAD, D), jnp.float32),
                  jax.ShapeDtypeStruct((2, HROWS, 128), jnp.float32)],
        scratch_types=[pltpu.VMEM((W_WIN // 2, D), jnp.float32),
                       pltpu.VMEM((N_PAD,), jnp.float32),
                       pltpu.VMEM((N_PAD,), jnp.float32),
                       pltpu.VMEM((N_PAD,), jnp.float32),
                       pltpu.VMEM((H,), jnp.float32),
                       pltpu.VMEM((HROWS, 128), jnp.float32),
                       pltpu.VMEM((HROWS,), jnp.int32),
                       pltpu.VMEM_SHARED((HROWS, 128), jnp.float32)],
        compiler_params=_sc_compiler_params(),
    )
    def kern(ta_hbm, tb_hbm, xp0_hbm, xp1_hbm, xp2_hbm, wd_hbm,
             row_hbm, col_hbm, u_hbm, cnt_hbm,
             bbuf, xb0, xb1, xb2, wdbuf, hist, ibuf, cnt_sh):
        cid = lax.axis_index("core")
        sid = lax.axis_index("subcore")
        zero16 = jnp.zeros((16,), jnp.float32)
        one16 = jnp.ones((16,), jnp.float32)
        iota16 = lax.iota(jnp.int32, 16)

        pltpu.sync_copy(xp0_hbm, xb0)
        pltpu.sync_copy(xp1_hbm, xb1)
        pltpu.sync_copy(xp2_hbm, xb2)
        pltpu.sync_copy(wd_hbm, wdbuf)

        @pl.loop(0, HROWS)
        def _(i):
            @pl.loop(0, 128, step=16)
            def _(c):
                hist[i, pl.ds(c, 16)] = zero16

        @pl.loop(0, HROWS, step=16)
        def _(g):
            ibuf[pl.ds(g, 16)] = iota16 + g

        @pl.when(sid < 10)
        def _():
            pltpu.sync_copy(hist.at[pl.ds(0, 8)],
                            cnt_sh.at[pl.ds(sid * 8, 8)])

        plsc.subcore_barrier()

        magic = jnp.full((16,), 0x5F3759DF, jnp.int32)

        def body(r_vmem, c_vmem, u_vmem):
            pltpu.sync_copy(ta_hbm.at[r_vmem.at[0]], u_vmem)
            for half in range(2):
                pltpu.sync_copy(
                    tb_hbm.at[c_vmem.at[0, pl.ds(half * 64, 64)]], bbuf)

                @pl.loop(0, 64)
                def _(e):
                    @pl.loop(0, D, step=16)
                    def _(c):
                        u_vmem[half * 64 + e, pl.ds(c, 16)] = (
                            u_vmem[half * 64 + e, pl.ds(c, 16)]
                            + bbuf[e, pl.ds(c, 16)])

            @pl.loop(0, W_WIN, step=16)
            def _(g):
                r16 = r_vmem[0, pl.ds(g, 16)]
                c16 = c_vmem[0, pl.ds(g, 16)]
                dx = (plsc.load_gather(xb0, [r16])
                      - plsc.load_gather(xb0, [c16]))
                dy = (plsc.load_gather(xb1, [r16])
                      - plsc.load_gather(xb1, [c16]))
                dz = (plsc.load_gather(xb2, [r16])
                      - plsc.load_gather(xb2, [c16]))
                v = dx * dx + dy * dy + dz * dz
                y = plsc.bitcast(magic - (plsc.bitcast(v, jnp.int32) >> 1),
                                 jnp.float32)
                y = y * (1.5 - 0.5 * v * y * y)
                y = y * (1.5 - 0.5 * v * y * y)
                y = y * (1.5 - 0.5 * v * y * y)
                dist16 = v * y
                plsc.addupdate_scatter(hist, [c16 >> 7, c16 & 127], one16)
                for lane in range(16):
                    s = dist16[lane]
                    for c in range(0, D, 16):
                        u_vmem[g + lane, pl.ds(c, 16)] = (
                            u_vmem[g + lane, pl.ds(c, 16)]
                            + s * wdbuf[pl.ds(c, 16)])

        pltpu.emit_pipeline(
            body,
            grid=(NWIN,),
            in_specs=[pl.BlockSpec((1, W_WIN), lambda i: (0, i)),
                      pl.BlockSpec((1, W_WIN), lambda i: (0, i))],
            out_specs=[pl.BlockSpec((W_WIN, D), lambda i: (i, 0))],
            core_axis_name=("core", "subcore"),
            dimension_semantics=(pltpu.PARALLEL,),
        )(row_hbm, col_hbm, u_hbm)

        plsc.subcore_barrier()
        pltpu.sync_copy(hist, cnt_sh.at[ibuf], add=True)
        plsc.subcore_barrier()

        @pl.when(sid < 10)
        def _():
            pltpu.sync_copy(cnt_sh.at[pl.ds(sid * 8, 8)],
                            cnt_hbm.at[cid, pl.ds(sid * 8, 8)])

    return kern(ta, tb, xp0, xp1, xp2, wd1, row2d, col2d)


def _edge_mlp_body(u_ref, we2_ref, be2_ref, m_ref):
    t = _silu(u_ref[...])
    m_ref[...] = _silu(jnp.dot(t, we2_ref[...],
                               preferred_element_type=jnp.float32)
                       + be2_ref[...])


def _edge_mlp(u, we2, be2):
    blk = 4096
    grid = (E_PAD // blk,)
    return pl.pallas_call(
        _edge_mlp_body,
        grid=grid,
        in_specs=[pl.BlockSpec((blk, D), lambda i: (i, 0)),
                  pl.BlockSpec((H, H), lambda i: (0, 0)),
                  pl.BlockSpec((1, H), lambda i: (0, 0))],
        out_specs=pl.BlockSpec((blk, H), lambda i: (i, 0)),
        out_shape=jax.ShapeDtypeStruct((E_PAD, H), jnp.float32),
    )(u, we2, be2)


def _inv_body(c_ref, inv_ref):
    c = c_ref[0] + c_ref[1]
    inv_ref[...] = (1.0 / jnp.maximum(c, 1.0)).reshape(N_PAD)


def _inv_counts(cnt):
    return pl.pallas_call(
        _inv_body,
        in_specs=[pl.BlockSpec((2, HROWS, 128), lambda: (0, 0, 0))],
        out_specs=pl.BlockSpec((N_PAD,), lambda: (0,)),
        out_shape=jax.ShapeDtypeStruct((N_PAD,), jnp.float32),
    )(cnt)


def _sc_scatter(m, col2d, inv):
    mesh = plsc.VectorSubcoreMesh(core_axis_name="core",
                                  subcore_axis_name="subcore")

    @functools.partial(
        pl.kernel,
        mesh=mesh,
        out_type=jax.ShapeDtypeStruct((2, N_PAD, H), jnp.float32),
        scratch_types=[pltpu.VMEM_SHARED((N_PAD, H), jnp.float32),
                       pltpu.VMEM((64, H), jnp.float32),
                       pltpu.VMEM((64,), jnp.float32)],
        compiler_params=_sc_compiler_params(),
    )
    def kern(m_hbm, col_hbm, inv_hbm, sums_out, sums_sh, zbuf, ivb):
        cid = lax.axis_index("core")
        sid = lax.axis_index("subcore")
        zero16 = jnp.zeros((16,), jnp.float32)

        @pl.loop(0, 64)
        def _(i):
            @pl.loop(0, H, step=16)
            def _(c):
                zbuf[i, pl.ds(c, 16)] = zero16

        base = sid * ROWS_PER_SUB

        @pl.loop(0, ROWS_PER_SUB // 64)
        def _(j):
            pltpu.sync_copy(zbuf, sums_sh.at[pl.ds(base + j * 64, 64)])

        plsc.subcore_barrier()

        def body(m_vmem, c_vmem):
            pltpu.sync_copy(m_vmem, sums_sh.at[c_vmem.at[0]], add=True)

        pltpu.emit_pipeline(
            body,
            grid=(NWIN,),
            in_specs=[pl.BlockSpec((W_WIN, H), lambda i: (i, 0)),
                      pl.BlockSpec((1, W_WIN), lambda i: (0, i))],
            out_specs=[],
            core_axis_name=("core", "subcore"),
            dimension_semantics=(pltpu.PARALLEL,),
        )(m_hbm, col_hbm)

        plsc.subcore_barrier()

        @pl.loop(0, ROWS_PER_SUB // 64)
        def _(j):
            pltpu.sync_copy(sums_sh.at[pl.ds(base + j * 64, 64)], zbuf)
            pltpu.sync_copy(inv_hbm.at[pl.ds(base + j * 64, 64)], ivb)
            for g in range(4):
                iv16 = ivb[pl.ds(g * 16, 16)]
                for lane in range(16):
                    s = iv16[lane]
                    for c in range(0, H, 16):
                        zbuf[g * 16 + lane, pl.ds(c, 16)] = (
                            zbuf[g * 16 + lane, pl.ds(c, 16)] * s)
            pltpu.sync_copy(zbuf, sums_out.at[cid, pl.ds(base + j * 64, 64)])

    return kern(m, col2d, inv)


def _node_final_body(h_ref, hn_ref, s0_ref, s1_ref,
                     wn1h_ref, wn1m_ref, bn1_ref, wn2_ref, bn2_ref,
                     wm1_ref, bm1_ref, wm2_ref, bm2_ref, g2_ref, b2_ref,
                     out_ref):
    m_aggr = s0_ref[...] + s1_ref[...]
    hn = hn_ref[...]
    z = _silu(jnp.dot(hn, wn1h_ref[...], preferred_element_type=jnp.float32)
              + jnp.dot(m_aggr, wn1m_ref[...],
                        preferred_element_type=jnp.float32)
              + bn1_ref[...])
    h_delta = jnp.dot(z, wn2_ref[...],
                      preferred_element_type=jnp.float32) + bn2_ref[...]
    h1 = h_ref[...] + hn + h_delta
    mu = jnp.mean(h1, axis=-1, keepdims=True)
    var = jnp.mean((h1 - mu) ** 2, axis=-1, keepdims=True)
    h2n = (h1 - mu) / jnp.sqrt(var + 1e-5) * g2_ref[...] + b2_ref[...]
    h_mlp = jnp.dot(_silu(jnp.dot(h2n, wm1_ref[...],
                                  preferred_element_type=jnp.float32)
                          + bm1_ref[...]),
                    wm2_ref[...], preferred_element_type=jnp.float32) \
        + bm2_ref[...]
    out_ref[...] = h1 + h_mlp


def _node_final(h, hn, s0, s1, wn1h, wn1m, bn1, wn2, bn2,
                wm1, bm1, wm2, bm2, g2, b2):
    blk = 1000
    grid = (N // blk,)
    rowspec = pl.BlockSpec((blk, D), lambda i: (i, 0))
    wspec = pl.BlockSpec((D, H), lambda i: (0, 0))
    vspec = pl.BlockSpec((1, H), lambda i: (0, 0))
    return pl.pallas_call(
        _node_final_body,
        grid=grid,
        in_specs=[rowspec, rowspec, rowspec, rowspec,
                  wspec, wspec, vspec, wspec, vspec,
                  wspec, vspec, wspec, vspec, vspec, vspec],
        out_specs=rowspec,
        out_shape=jax.ShapeDtypeStruct((N, D), jnp.float32),
    )(h, hn, s0, s1, wn1h, wn1m, bn1, wn2, bn2,
      wm1, bm1, wm2, bm2, g2, b2)


def kernel(x, h, edge_index, W_e1, b_e1, W_e2, b_e2, W_n1, b_n1, W_n2, b_n2,
           W_m1, b_m1, W_m2, b_m2, g1, beta1, g2, beta2):
    row = edge_index[0].astype(jnp.int32)
    col = edge_index[1].astype(jnp.int32)

    w1a = W_e1[:D]
    w1b = W_e1[D:2 * D]
    wd = W_e1[2 * D]

    hn, a_tab, b_tab = _node_prep(h, w1a, w1b, b_e1.reshape(1, H),
                                  g1.reshape(1, D), beta1.reshape(1, D))

    ta = jnp.pad(a_tab, ((0, N_PAD - N), (0, 0)))
    tb = jnp.pad(b_tab, ((0, N_PAD - N), (0, 0)))
    xpad = jnp.pad(x, ((0, N_PAD - N), (0, 0)))
    xp0, xp1, xp2 = xpad[:, 0], xpad[:, 1], xpad[:, 2]

    pad = E_PAD - E
    row_p = jnp.concatenate([row, jnp.zeros((pad,), jnp.int32)]).reshape(
        1, E_PAD)
    col_p = jnp.concatenate([col, jnp.full((pad,), DUMMY, jnp.int32)]
                            ).reshape(1, E_PAD)

    u, cnt_p = _sc_gather(ta, tb, xp0, xp1, xp2, wd, row_p, col_p)

    m = _edge_mlp(u, W_e2, b_e2.reshape(1, H))
    inv = _inv_counts(cnt_p)

    sums_p = _sc_scatter(m, col_p, inv)

    out = _node_final(h, hn,
                      sums_p[0, :N], sums_p[1, :N],
                      W_n1[:D], W_n1[D:], b_n1.reshape(1, H),
                      W_n2, b_n2.reshape(1, D),
                      W_m1, b_m1.reshape(1, H),
                      W_m2, b_m2.reshape(1, D),
                      g2.reshape(1, D), beta2.reshape(1, D))
    return out


# speedup vs baseline: 3.3434x; 1.3372x over previous
import dataclasses
import functools

import jax
import jax.numpy as jnp
from jax import lax
from jax.experimental import pallas as pl
from jax.experimental.pallas import tpu as pltpu
from jax.experimental.pallas import tpu_sc as plsc

N = 10000
E = 320000
D = 128
H = 128

W_WIN = 128
N_WORKERS = 32
WINS = 79
E_PAD = N_WORKERS * WINS * W_WIN
NWIN = E_PAD // W_WIN
D2R = E_PAD // 128
N_PAD = 10240
DUMMY = N
ROWS_PER_SUB = N_PAD // 16
HROWS = 80


def _silu(v):
    return v * jax.nn.sigmoid(v)


def _sc_compiler_params():
    cp = pltpu.CompilerParams()
    if "needs_layout_passes" in pltpu.CompilerParams.__dataclass_fields__:
        cp = dataclasses.replace(cp, needs_layout_passes=False)
    return cp


def _node_prep_body(h_ref, w1a_ref, w1b_ref, be1_ref, g1_ref, b1_ref,
                    hn_ref, a_ref, b_ref):
    h = h_ref[...]
    mu = jnp.mean(h, axis=-1, keepdims=True)
    var = jnp.mean((h - mu) ** 2, axis=-1, keepdims=True)
    hn = (h - mu) / jnp.sqrt(var + 1e-5) * g1_ref[...] + b1_ref[...]
    hn_ref[...] = hn
    a_ref[...] = jnp.dot(hn, w1a_ref[...],
                         preferred_element_type=jnp.float32) + be1_ref[...]
    b_ref[...] = jnp.dot(hn, w1b_ref[...], preferred_element_type=jnp.float32)


def _node_prep(h, w1a, w1b, be1, g1, b1):
    blk = 1000
    grid = (N // blk,)
    rowspec = pl.BlockSpec((blk, D), lambda i: (i, 0))
    wspec = pl.BlockSpec((D, H), lambda i: (0, 0))
    vspec = pl.BlockSpec((1, H), lambda i: (0, 0))
    return pl.pallas_call(
        _node_prep_body,
        grid=grid,
        in_specs=[rowspec, wspec, wspec, vspec, vspec, vspec],
        out_specs=[rowspec, rowspec, rowspec],
        out_shape=[jax.ShapeDtypeStruct((N, D), jnp.float32)] * 3,
    )(h, w1a, w1b, be1, g1, b1)


def _sc_gather(ta, tb, xp0, xp1, xp2, row2d, col2d):
    mesh = plsc.VectorSubcoreMesh(core_axis_name="core",
                                  subcore_axis_name="subcore")

    @functools.partial(
        pl.kernel,
        mesh=mesh,
        out_type=[jax.ShapeDtypeStruct((E_PAD, D), jnp.float32),
                  jax.ShapeDtypeStruct((D2R, 128), jnp.float32),
                  jax.ShapeDtypeStruct((2, HROWS, 128), jnp.float32)],
        scratch_types=[pltpu.VMEM((W_WIN // 2, D), jnp.float32),
                       pltpu.VMEM((N_PAD,), jnp.float32),
                       pltpu.VMEM((N_PAD,), jnp.float32),
                       pltpu.VMEM((N_PAD,), jnp.float32),
                       pltpu.VMEM((HROWS, 128), jnp.float32),
                       pltpu.VMEM((HROWS,), jnp.int32),
                       pltpu.VMEM_SHARED((HROWS, 128), jnp.float32)],
        compiler_params=_sc_compiler_params(),
    )
    def kern(ta_hbm, tb_hbm, xp0_hbm, xp1_hbm, xp2_hbm,
             row_hbm, col_hbm, u_hbm, d2_hbm, cnt_hbm,
             bbuf, xb0, xb1, xb2, hist, ibuf, cnt_sh):
        cid = lax.axis_index("core")
        sid = lax.axis_index("subcore")
        zero16 = jnp.zeros((16,), jnp.float32)
        one16 = jnp.ones((16,), jnp.float32)
        iota16 = lax.iota(jnp.int32, 16)

        pltpu.sync_copy(xp0_hbm, xb0)
        pltpu.sync_copy(xp1_hbm, xb1)
        pltpu.sync_copy(xp2_hbm, xb2)

        @pl.loop(0, HROWS)
        def _(i):
            @pl.loop(0, 128, step=16)
            def _(c):
                hist[i, pl.ds(c, 16)] = zero16

        @pl.loop(0, HROWS, step=16)
        def _(g):
            ibuf[pl.ds(g, 16)] = iota16 + g

        @pl.when(sid < 10)
        def _():
            pltpu.sync_copy(hist.at[pl.ds(0, 8)],
                            cnt_sh.at[pl.ds(sid * 8, 8)])

        plsc.subcore_barrier()

        def body(r_vmem, c_vmem, u_vmem, d2_vmem):
            pltpu.sync_copy(ta_hbm.at[r_vmem.at[0]], u_vmem)
            for half in range(2):
                pltpu.sync_copy(
                    tb_hbm.at[c_vmem.at[0, pl.ds(half * 64, 64)]], bbuf)

                @pl.loop(0, 64, unroll=4)
                def _(e):
                    for c in range(0, D, 16):
                        u_vmem[half * 64 + e, pl.ds(c, 16)] = (
                            u_vmem[half * 64 + e, pl.ds(c, 16)]
                            + bbuf[e, pl.ds(c, 16)])

            @pl.loop(0, W_WIN, step=16, unroll=2)
            def _(g):
                r16 = r_vmem[0, pl.ds(g, 16)]
                c16 = c_vmem[0, pl.ds(g, 16)]
                dx = (plsc.load_gather(xb0, [r16])
                      - plsc.load_gather(xb0, [c16]))
                dy = (plsc.load_gather(xb1, [r16])
                      - plsc.load_gather(xb1, [c16]))
                dz = (plsc.load_gather(xb2, [r16])
                      - plsc.load_gather(xb2, [c16]))
                d2_vmem[0, pl.ds(g, 16)] = dx * dx + dy * dy + dz * dz
                plsc.addupdate_scatter(hist, [c16 >> 7, c16 & 127], one16)

        pltpu.emit_pipeline(
            body,
            grid=(NWIN,),
            in_specs=[pl.BlockSpec((1, W_WIN), lambda i: (0, i)),
                      pl.BlockSpec((1, W_WIN), lambda i: (0, i))],
            out_specs=[pl.BlockSpec((W_WIN, D), lambda i: (i, 0)),
                       pl.BlockSpec((1, W_WIN), lambda i: (i, 0))],
            core_axis_name=("core", "subcore"),
            dimension_semantics=(pltpu.PARALLEL,),
        )(row_hbm, col_hbm, u_hbm, d2_hbm)

        plsc.subcore_barrier()
        pltpu.sync_copy(hist, cnt_sh.at[ibuf], add=True)
        plsc.subcore_barrier()

        @pl.when(sid < 10)
        def _():
            pltpu.sync_copy(cnt_sh.at[pl.ds(sid * 8, 8)],
                            cnt_hbm.at[cid, pl.ds(sid * 8, 8)])

    return kern(ta, tb, xp0, xp1, xp2, row2d, col2d)


def _edge_mlp_body(u_ref, d2_ref, wd_ref, we2_ref, be2_ref, m_ref):
    nrow = d2_ref.shape[0]
    dist = jnp.sqrt(d2_ref[...])
    eye = (lax.broadcasted_iota(jnp.int32, (128, 128), 0)
           == lax.broadcasted_iota(jnp.int32, (128, 128), 1)
           ).astype(jnp.float32)
    dt = lax.dot_general(eye, dist, (((1,), (1,)), ((), ())),
                         preferred_element_type=jnp.float32)
    u = u_ref[...]
    wd = wd_ref[...]
    chunks = []
    for r in range(nrow):
        chunks.append(u[128 * r:128 * (r + 1), :] + dt[:, r:r + 1] * wd)
    t = _silu(jnp.concatenate(chunks, axis=0))
    m_ref[...] = _silu(jnp.dot(t, we2_ref[...],
                               preferred_element_type=jnp.float32)
                       + be2_ref[...])


def _edge_mlp(u, d2, wd, we2, be2):
    blk = 4096
    grid = (E_PAD // blk,)
    return pl.pallas_call(
        _edge_mlp_body,
        grid=grid,
        in_specs=[pl.BlockSpec((blk, D), lambda i: (i, 0)),
                  pl.BlockSpec((blk // 128, 128), lambda i: (i, 0)),
                  pl.BlockSpec((1, H), lambda i: (0, 0)),
                  pl.BlockSpec((H, H), lambda i: (0, 0)),
                  pl.BlockSpec((1, H), lambda i: (0, 0))],
        out_specs=pl.BlockSpec((blk, H), lambda i: (i, 0)),
        out_shape=jax.ShapeDtypeStruct((E_PAD, H), jnp.float32),
    )(u, d2, wd, we2, be2)


def _inv_body(c_ref, inv_ref):
    c = c_ref[0] + c_ref[1]
    inv_ref[...] = (1.0 / jnp.maximum(c, 1.0)).reshape(N_PAD)


def _inv_counts(cnt):
    return pl.pallas_call(
        _inv_body,
        in_specs=[pl.BlockSpec((2, HROWS, 128), lambda: (0, 0, 0))],
        out_specs=pl.BlockSpec((N_PAD,), lambda: (0,)),
        out_shape=jax.ShapeDtypeStruct((N_PAD,), jnp.float32),
    )(cnt)


def _sc_scatter(m, col2d, inv):
    mesh = plsc.VectorSubcoreMesh(core_axis_name="core",
                                  subcore_axis_name="subcore")

    @functools.partial(
        pl.kernel,
        mesh=mesh,
        out_type=jax.ShapeDtypeStruct((2, N_PAD, H), jnp.float32),
        scratch_types=[pltpu.VMEM_SHARED((N_PAD, H), jnp.float32),
                       pltpu.VMEM((64, H), jnp.float32),
                       pltpu.VMEM((64,), jnp.float32)],
        compiler_params=_sc_compiler_params(),
    )
    def kern(m_hbm, col_hbm, inv_hbm, sums_out, sums_sh, zbuf, ivb):
        cid = lax.axis_index("core")
        sid = lax.axis_index("subcore")
        zero16 = jnp.zeros((16,), jnp.float32)

        @pl.loop(0, 64)
        def _(i):
            @pl.loop(0, H, step=16)
            def _(c):
                zbuf[i, pl.ds(c, 16)] = zero16

        base = sid * ROWS_PER_SUB

        @pl.loop(0, ROWS_PER_SUB // 64)
        def _(j):
            pltpu.sync_copy(zbuf, sums_sh.at[pl.ds(base + j * 64, 64)])

        plsc.subcore_barrier()

        def body(m_vmem, c_vmem):
            pltpu.sync_copy(m_vmem, sums_sh.at[c_vmem.at[0]], add=True)

        pltpu.emit_pipeline(
            body,
            grid=(NWIN,),
            in_specs=[pl.BlockSpec((W_WIN, H), lambda i: (i, 0)),
                      pl.BlockSpec((1, W_WIN), lambda i: (0, i))],
            out_specs=[],
            core_axis_name=("core", "subcore"),
            dimension_semantics=(pltpu.PARALLEL,),
        )(m_hbm, col_hbm)

        plsc.subcore_barrier()

        @pl.loop(0, ROWS_PER_SUB // 64)
        def _(j):
            pltpu.sync_copy(sums_sh.at[pl.ds(base + j * 64, 64)], zbuf)
            pltpu.sync_copy(inv_hbm.at[pl.ds(base + j * 64, 64)], ivb)
            for g in range(4):
                iv16 = ivb[pl.ds(g * 16, 16)]
                for lane in range(16):
                    s = iv16[lane]
                    for c in range(0, H, 16):
                        zbuf[g * 16 + lane, pl.ds(c, 16)] = (
                            zbuf[g * 16 + lane, pl.ds(c, 16)] * s)
            pltpu.sync_copy(zbuf, sums_out.at[cid, pl.ds(base + j * 64, 64)])

    return kern(m, col2d, inv)


def _node_final_body(h_ref, hn_ref, s0_ref, s1_ref,
                     wn1h_ref, wn1m_ref, bn1_ref, wn2_ref, bn2_ref,
                     wm1_ref, bm1_ref, wm2_ref, bm2_ref, g2_ref, b2_ref,
                     out_ref):
    m_aggr = s0_ref[...] + s1_ref[...]
    hn = hn_ref[...]
    z = _silu(jnp.dot(hn, wn1h_ref[...], preferred_element_type=jnp.float32)
              + jnp.dot(m_aggr, wn1m_ref[...],
                        preferred_element_type=jnp.float32)
              + bn1_ref[...])
    h_delta = jnp.dot(z, wn2_ref[...],
                      preferred_element_type=jnp.float32) + bn2_ref[...]
    h1 = h_ref[...] + hn + h_delta
    mu = jnp.mean(h1, axis=-1, keepdims=True)
    var = jnp.mean((h1 - mu) ** 2, axis=-1, keepdims=True)
    h2n = (h1 - mu) / jnp.sqrt(var + 1e-5) * g2_ref[...] + b2_ref[...]
    h_mlp = jnp.dot(_silu(jnp.dot(h2n, wm1_ref[...],
                                  preferred_element_type=jnp.float32)
                          + bm1_ref[...]),
                    wm2_ref[...], preferred_element_type=jnp.float32) \
        + bm2_ref[...]
    out_ref[...] = h1 + h_mlp


def _node_final(h, hn, s0, s1, wn1h, wn1m, bn1, wn2, bn2,
                wm1, bm1, wm2, bm2, g2, b2):
    blk = 1000
    grid = (N // blk,)
    rowspec = pl.BlockSpec((blk, D), lambda i: (i, 0))
    wspec = pl.BlockSpec((D, H), lambda i: (0, 0))
    vspec = pl.BlockSpec((1, H), lambda i: (0, 0))
    return pl.pallas_call(
        _node_final_body,
        grid=grid,
        in_specs=[rowspec, rowspec, rowspec, rowspec,
                  wspec, wspec, vspec, wspec, vspec,
                  wspec, vspec, wspec, vspec, vspec, vspec],
        out_specs=rowspec,
        out_shape=jax.ShapeDtypeStruct((N, D), jnp.float32),
    )(h, hn, s0, s1, wn1h, wn1m, bn1, wn2, bn2,
      wm1, bm1, wm2, bm2, g2, b2)


def kernel(x, h, edge_index, W_e1, b_e1, W_e2, b_e2, W_n1, b_n1, W_n2, b_n2,
           W_m1, b_m1, W_m2, b_m2, g1, beta1, g2, beta2):
    row = edge_index[0].astype(jnp.int32)
    col = edge_index[1].astype(jnp.int32)

    w1a = W_e1[:D]
    w1b = W_e1[D:2 * D]
    wd = W_e1[2 * D]

    hn, a_tab, b_tab = _node_prep(h, w1a, w1b, b_e1.reshape(1, H),
                                  g1.reshape(1, D), beta1.reshape(1, D))

    ta = jnp.pad(a_tab, ((0, N_PAD - N), (0, 0)))
    tb = jnp.pad(b_tab, ((0, N_PAD - N), (0, 0)))
    xpad = jnp.pad(x, ((0, N_PAD - N), (0, 0)))
    xp0, xp1, xp2 = xpad[:, 0], xpad[:, 1], xpad[:, 2]

    pad = E_PAD - E
    row_p = jnp.concatenate([row, jnp.zeros((pad,), jnp.int32)]).reshape(
        1, E_PAD)
    col_p = jnp.concatenate([col, jnp.full((pad,), DUMMY, jnp.int32)]
                            ).reshape(1, E_PAD)

    u, d2, cnt_p = _sc_gather(ta, tb, xp0, xp1, xp2, row_p, col_p)

    m = _edge_mlp(u, d2, wd.reshape(1, H), W_e2, b_e2.reshape(1, H))
    inv = _inv_counts(cnt_p)

    sums_p = _sc_scatter(m, col_p, inv)

    out = _node_final(h, hn,
                      sums_p[0, :N], sums_p[1, :N],
                      W_n1[:D], W_n1[D:], b_n1.reshape(1, H),
                      W_n2, b_n2.reshape(1, D),
                      W_m1, b_m1.reshape(1, H),
                      W_m2, b_m2.reshape(1, D),
                      g2.reshape(1, D), beta2.reshape(1, D))
    return out


# speedup vs baseline: 4.2824x; 1.2808x over previous
import dataclasses
import functools

import jax
import jax.numpy as jnp
from jax import lax
from jax.experimental import pallas as pl
from jax.experimental.pallas import tpu as pltpu
from jax.experimental.pallas import tpu_sc as plsc

N = 10000
E = 320000
D = 128
H = 128

W_WIN = 128
N_WORKERS = 32
WINS = 79
E_PAD = N_WORKERS * WINS * W_WIN
NWIN = E_PAD // W_WIN
D2R = E_PAD // 128
N_PAD = 10240
DUMMY = N
ROWS_PER_SUB = N_PAD // 16
HROWS = 80


def _silu(v):
    return v * jax.nn.sigmoid(v)


def _sc_compiler_params():
    cp = pltpu.CompilerParams()
    if "needs_layout_passes" in pltpu.CompilerParams.__dataclass_fields__:
        cp = dataclasses.replace(cp, needs_layout_passes=False)
    return cp


def _node_prep_body(h_ref, w1a_ref, w1b_ref, be1_ref, g1_ref, b1_ref,
                    hn_ref, a_ref, b_ref):
    h = h_ref[...]
    mu = jnp.mean(h, axis=-1, keepdims=True)
    var = jnp.mean((h - mu) ** 2, axis=-1, keepdims=True)
    hn = (h - mu) / jnp.sqrt(var + 1e-5) * g1_ref[...] + b1_ref[...]
    hn_ref[...] = hn
    a_ref[...] = jnp.dot(hn, w1a_ref[...],
                         preferred_element_type=jnp.float32) + be1_ref[...]
    b_ref[...] = jnp.dot(hn, w1b_ref[...], preferred_element_type=jnp.float32)


def _node_prep(h, w1a, w1b, be1, g1, b1):
    blk = 1000
    grid = (N // blk,)
    rowspec = pl.BlockSpec((blk, D), lambda i: (i, 0))
    wspec = pl.BlockSpec((D, H), lambda i: (0, 0))
    vspec = pl.BlockSpec((1, H), lambda i: (0, 0))
    return pl.pallas_call(
        _node_prep_body,
        grid=grid,
        in_specs=[rowspec, wspec, wspec, vspec, vspec, vspec],
        out_specs=[rowspec, rowspec, rowspec],
        out_shape=[jax.ShapeDtypeStruct((N, D), jnp.float32)] * 3,
    )(h, w1a, w1b, be1, g1, b1)


def _sc_gather(ta, tb, xp0, xp1, xp2, row2d, col2d):
    mesh = plsc.VectorSubcoreMesh(core_axis_name="core",
                                  subcore_axis_name="subcore")

    @functools.partial(
        pl.kernel,
        mesh=mesh,
        out_type=[jax.ShapeDtypeStruct((E_PAD, D), jnp.float32),
                  jax.ShapeDtypeStruct((D2R, 128), jnp.float32),
                  jax.ShapeDtypeStruct((2, HROWS, 128), jnp.float32)],
        scratch_types=[pltpu.VMEM((W_WIN // 2, D), jnp.float32),
                       pltpu.VMEM((W_WIN // 2, D), jnp.float32),
                       pltpu.VMEM((N_PAD,), jnp.float32),
                       pltpu.VMEM((N_PAD,), jnp.float32),
                       pltpu.VMEM((N_PAD,), jnp.float32),
                       pltpu.VMEM((HROWS, 128), jnp.float32),
                       pltpu.VMEM((HROWS,), jnp.int32),
                       pltpu.VMEM_SHARED((HROWS, 128), jnp.float32),
                       pltpu.SemaphoreType.DMA,
                       pltpu.SemaphoreType.DMA,
                       pltpu.SemaphoreType.DMA],
        compiler_params=_sc_compiler_params(),
    )
    def kern(ta_hbm, tb_hbm, xp0_hbm, xp1_hbm, xp2_hbm,
             row_hbm, col_hbm, u_hbm, d2_hbm, cnt_hbm,
             bbuf, bbuf2, xb0, xb1, xb2, hist, ibuf, cnt_sh,
             semA, semB, semC):
        cid = lax.axis_index("core")
        sid = lax.axis_index("subcore")
        zero16 = jnp.zeros((16,), jnp.float32)
        one16 = jnp.ones((16,), jnp.float32)
        iota16 = lax.iota(jnp.int32, 16)

        pltpu.sync_copy(xp0_hbm, xb0)
        pltpu.sync_copy(xp1_hbm, xb1)
        pltpu.sync_copy(xp2_hbm, xb2)

        @pl.loop(0, HROWS)
        def _(i):
            @pl.loop(0, 128, step=16)
            def _(c):
                hist[i, pl.ds(c, 16)] = zero16

        @pl.loop(0, HROWS, step=16)
        def _(g):
            ibuf[pl.ds(g, 16)] = iota16 + g

        @pl.when(sid < 10)
        def _():
            pltpu.sync_copy(hist.at[pl.ds(0, 8)],
                            cnt_sh.at[pl.ds(sid * 8, 8)])

        plsc.subcore_barrier()

        def body(r_vmem, c_vmem, u_vmem, d2_vmem):
            hA = pltpu.async_copy(ta_hbm.at[r_vmem.at[0]], u_vmem, semA)
            hB0 = pltpu.async_copy(
                tb_hbm.at[c_vmem.at[0, pl.ds(0, 64)]], bbuf, semB)
            hB1 = pltpu.async_copy(
                tb_hbm.at[c_vmem.at[0, pl.ds(64, 64)]], bbuf2, semC)

            @pl.loop(0, W_WIN, step=16, unroll=2)
            def _(g):
                r16 = r_vmem[0, pl.ds(g, 16)]
                c16 = c_vmem[0, pl.ds(g, 16)]
                dx = (plsc.load_gather(xb0, [r16])
                      - plsc.load_gather(xb0, [c16]))
                dy = (plsc.load_gather(xb1, [r16])
                      - plsc.load_gather(xb1, [c16]))
                dz = (plsc.load_gather(xb2, [r16])
                      - plsc.load_gather(xb2, [c16]))
                d2_vmem[0, pl.ds(g, 16)] = dx * dx + dy * dy + dz * dz
                plsc.addupdate_scatter(hist, [c16 >> 7, c16 & 127], one16)

            hA.wait()
            hB0.wait()

            @pl.loop(0, 64, unroll=4)
            def _(e):
                for c in range(0, D, 16):
                    u_vmem[e, pl.ds(c, 16)] = (u_vmem[e, pl.ds(c, 16)]
                                               + bbuf[e, pl.ds(c, 16)])

            hB1.wait()

            @pl.loop(0, 64, unroll=4)
            def _(e):
                for c in range(0, D, 16):
                    u_vmem[64 + e, pl.ds(c, 16)] = (
                        u_vmem[64 + e, pl.ds(c, 16)]
                        + bbuf2[e, pl.ds(c, 16)])

        pltpu.emit_pipeline(
            body,
            grid=(NWIN,),
            in_specs=[pl.BlockSpec((1, W_WIN), lambda i: (0, i)),
                      pl.BlockSpec((1, W_WIN), lambda i: (0, i))],
            out_specs=[pl.BlockSpec((W_WIN, D), lambda i: (i, 0)),
                       pl.BlockSpec((1, W_WIN), lambda i: (i, 0))],
            core_axis_name=("core", "subcore"),
            dimension_semantics=(pltpu.PARALLEL,),
        )(row_hbm, col_hbm, u_hbm, d2_hbm)

        plsc.subcore_barrier()
        pltpu.sync_copy(hist, cnt_sh.at[ibuf], add=True)
        plsc.subcore_barrier()

        @pl.when(sid < 10)
        def _():
            pltpu.sync_copy(cnt_sh.at[pl.ds(sid * 8, 8)],
                            cnt_hbm.at[cid, pl.ds(sid * 8, 8)])

    return kern(ta, tb, xp0, xp1, xp2, row2d, col2d)


def _edge_mlp_body(u_ref, d2_ref, wd_ref, we2_ref, be2_ref, m_ref):
    nrow = d2_ref.shape[0]
    dist = jnp.sqrt(d2_ref[...])
    eye = (lax.broadcasted_iota(jnp.int32, (128, 128), 0)
           == lax.broadcasted_iota(jnp.int32, (128, 128), 1)
           ).astype(jnp.float32)
    dt = lax.dot_general(eye, dist, (((1,), (1,)), ((), ())),
                         preferred_element_type=jnp.float32)
    u = u_ref[...]
    wd = wd_ref[...]
    chunks = []
    for r in range(nrow):
        chunks.append(u[128 * r:128 * (r + 1), :] + dt[:, r:r + 1] * wd)
    t = _silu(jnp.concatenate(chunks, axis=0))
    m_ref[...] = _silu(jnp.dot(t, we2_ref[...],
                               preferred_element_type=jnp.float32)
                       + be2_ref[...])


def _edge_mlp(u, d2, wd, we2, be2):
    blk = 4096
    grid = (E_PAD // blk,)
    return pl.pallas_call(
        _edge_mlp_body,
        grid=grid,
        in_specs=[pl.BlockSpec((blk, D), lambda i: (i, 0)),
                  pl.BlockSpec((blk // 128, 128), lambda i: (i, 0)),
                  pl.BlockSpec((1, H), lambda i: (0, 0)),
                  pl.BlockSpec((H, H), lambda i: (0, 0)),
                  pl.BlockSpec((1, H), lambda i: (0, 0))],
        out_specs=pl.BlockSpec((blk, H), lambda i: (i, 0)),
        out_shape=jax.ShapeDtypeStruct((E_PAD, H), jnp.float32),
    )(u, d2, wd, we2, be2)


def _inv_body(c_ref, inv_ref):
    c = c_ref[0] + c_ref[1]
    inv_ref[...] = (1.0 / jnp.maximum(c, 1.0)).reshape(N_PAD)


def _inv_counts(cnt):
    return pl.pallas_call(
        _inv_body,
        in_specs=[pl.BlockSpec((2, HROWS, 128), lambda: (0, 0, 0))],
        out_specs=pl.BlockSpec((N_PAD,), lambda: (0,)),
        out_shape=jax.ShapeDtypeStruct((N_PAD,), jnp.float32),
    )(cnt)


def _sc_scatter(m, col2d, inv):
    mesh = plsc.VectorSubcoreMesh(core_axis_name="core",
                                  subcore_axis_name="subcore")

    @functools.partial(
        pl.kernel,
        mesh=mesh,
        out_type=jax.ShapeDtypeStruct((2, N_PAD, H), jnp.float32),
        scratch_types=[pltpu.VMEM_SHARED((N_PAD, H), jnp.float32),
                       pltpu.VMEM((64, H), jnp.float32),
                       pltpu.VMEM((64,), jnp.float32)],
        compiler_params=_sc_compiler_params(),
    )
    def kern(m_hbm, col_hbm, inv_hbm, sums_out, sums_sh, zbuf, ivb):
        cid = lax.axis_index("core")
        sid = lax.axis_index("subcore")
        zero16 = jnp.zeros((16,), jnp.float32)

        @pl.loop(0, 64)
        def _(i):
            @pl.loop(0, H, step=16)
            def _(c):
                zbuf[i, pl.ds(c, 16)] = zero16

        base = sid * ROWS_PER_SUB

        @pl.loop(0, ROWS_PER_SUB // 64)
        def _(j):
            pltpu.sync_copy(zbuf, sums_sh.at[pl.ds(base + j * 64, 64)])

        plsc.subcore_barrier()

        def body(m_vmem, c_vmem):
            pltpu.sync_copy(m_vmem, sums_sh.at[c_vmem.at[0]], add=True)

        pltpu.emit_pipeline(
            body,
            grid=(NWIN,),
            in_specs=[pl.BlockSpec((W_WIN, H), lambda i: (i, 0)),
                      pl.BlockSpec((1, W_WIN), lambda i: (0, i))],
            out_specs=[],
            core_axis_name=("core", "subcore"),
            dimension_semantics=(pltpu.PARALLEL,),
        )(m_hbm, col_hbm)

        plsc.subcore_barrier()

        @pl.loop(0, ROWS_PER_SUB // 64)
        def _(j):
            pltpu.sync_copy(sums_sh.at[pl.ds(base + j * 64, 64)], zbuf)
            pltpu.sync_copy(inv_hbm.at[pl.ds(base + j * 64, 64)], ivb)
            for g in range(4):
                iv16 = ivb[pl.ds(g * 16, 16)]
                for lane in range(16):
                    s = iv16[lane]
                    for c in range(0, H, 16):
                        zbuf[g * 16 + lane, pl.ds(c, 16)] = (
                            zbuf[g * 16 + lane, pl.ds(c, 16)] * s)
            pltpu.sync_copy(zbuf, sums_out.at[cid, pl.ds(base + j * 64, 64)])

    return kern(m, col2d, inv)


def _node_final_body(h_ref, hn_ref, s0_ref, s1_ref,
                     wn1h_ref, wn1m_ref, bn1_ref, wn2_ref, bn2_ref,
                     wm1_ref, bm1_ref, wm2_ref, bm2_ref, g2_ref, b2_ref,
                     out_ref):
    m_aggr = s0_ref[...] + s1_ref[...]
    hn = hn_ref[...]
    z = _silu(jnp.dot(hn, wn1h_ref[...], preferred_element_type=jnp.float32)
              + jnp.dot(m_aggr, wn1m_ref[...],
                        preferred_element_type=jnp.float32)
              + bn1_ref[...])
    h_delta = jnp.dot(z, wn2_ref[...],
                      preferred_element_type=jnp.float32) + bn2_ref[...]
    h1 = h_ref[...] + hn + h_delta
    mu = jnp.mean(h1, axis=-1, keepdims=True)
    var = jnp.mean((h1 - mu) ** 2, axis=-1, keepdims=True)
    h2n = (h1 - mu) / jnp.sqrt(var + 1e-5) * g2_ref[...] + b2_ref[...]
    h_mlp = jnp.dot(_silu(jnp.dot(h2n, wm1_ref[...],
                                  preferred_element_type=jnp.float32)
                          + bm1_ref[...]),
                    wm2_ref[...], preferred_element_type=jnp.float32) \
        + bm2_ref[...]
    out_ref[...] = h1 + h_mlp


def _node_final(h, hn, s0, s1, wn1h, wn1m, bn1, wn2, bn2,
                wm1, bm1, wm2, bm2, g2, b2):
    blk = 1000
    grid = (N // blk,)
    rowspec = pl.BlockSpec((blk, D), lambda i: (i, 0))
    wspec = pl.BlockSpec((D, H), lambda i: (0, 0))
    vspec = pl.BlockSpec((1, H), lambda i: (0, 0))
    return pl.pallas_call(
        _node_final_body,
        grid=grid,
        in_specs=[rowspec, rowspec, rowspec, rowspec,
                  wspec, wspec, vspec, wspec, vspec,
                  wspec, vspec, wspec, vspec, vspec, vspec],
        out_specs=rowspec,
        out_shape=jax.ShapeDtypeStruct((N, D), jnp.float32),
    )(h, hn, s0, s1, wn1h, wn1m, bn1, wn2, bn2,
      wm1, bm1, wm2, bm2, g2, b2)


def kernel(x, h, edge_index, W_e1, b_e1, W_e2, b_e2, W_n1, b_n1, W_n2, b_n2,
           W_m1, b_m1, W_m2, b_m2, g1, beta1, g2, beta2):
    row = edge_index[0].astype(jnp.int32)
    col = edge_index[1].astype(jnp.int32)

    w1a = W_e1[:D]
    w1b = W_e1[D:2 * D]
    wd = W_e1[2 * D]

    hn, a_tab, b_tab = _node_prep(h, w1a, w1b, b_e1.reshape(1, H),
                                  g1.reshape(1, D), beta1.reshape(1, D))

    ta = jnp.pad(a_tab, ((0, N_PAD - N), (0, 0)))
    tb = jnp.pad(b_tab, ((0, N_PAD - N), (0, 0)))
    xpad = jnp.pad(x, ((0, N_PAD - N), (0, 0)))
    xp0, xp1, xp2 = xpad[:, 0], xpad[:, 1], xpad[:, 2]

    pad = E_PAD - E
    row_p = jnp.concatenate([row, jnp.zeros((pad,), jnp.int32)]).reshape(
        1, E_PAD)
    col_p = jnp.concatenate([col, jnp.full((pad,), DUMMY, jnp.int32)]
                            ).reshape(1, E_PAD)

    u, d2, cnt_p = _sc_gather(ta, tb, xp0, xp1, xp2, row_p, col_p)

    m = _edge_mlp(u, d2, wd.reshape(1, H), W_e2, b_e2.reshape(1, H))
    inv = _inv_counts(cnt_p)

    sums_p = _sc_scatter(m, col_p, inv)

    out = _node_final(h, hn,
                      sums_p[0, :N], sums_p[1, :N],
                      W_n1[:D], W_n1[D:], b_n1.reshape(1, H),
                      W_n2, b_n2.reshape(1, D),
                      W_m1, b_m1.reshape(1, H),
                      W_m2, b_m2.reshape(1, D),
                      g2.reshape(1, D), beta2.reshape(1, D))
    return out


# speedup vs baseline: 4.3751x; 1.0216x over previous
import dataclasses
import functools

import jax
import jax.numpy as jnp
from jax import lax
from jax.experimental import pallas as pl
from jax.experimental.pallas import tpu as pltpu
from jax.experimental.pallas import tpu_sc as plsc

N = 10000
E = 320000
D = 128
H = 128

W_WIN = 128
N_WORKERS = 32
WINS = 79
E_PAD = N_WORKERS * WINS * W_WIN
NWIN = E_PAD // W_WIN
D2R = E_PAD // 128
N_PAD = 10240
DUMMY = N
ROWS_PER_SUB = N_PAD // 16
HROWS = 80


def _silu(v):
    return v * jax.nn.sigmoid(v)


def _sc_compiler_params():
    cp = pltpu.CompilerParams()
    if "needs_layout_passes" in pltpu.CompilerParams.__dataclass_fields__:
        cp = dataclasses.replace(cp, needs_layout_passes=False)
    return cp


def _node_prep_body(h_ref, w1a_ref, w1b_ref, be1_ref, g1_ref, b1_ref,
                    hn_ref, a_ref, b_ref):
    h = h_ref[...]
    mu = jnp.mean(h, axis=-1, keepdims=True)
    var = jnp.mean((h - mu) ** 2, axis=-1, keepdims=True)
    hn = (h - mu) / jnp.sqrt(var + 1e-5) * g1_ref[...] + b1_ref[...]
    hn_ref[...] = hn
    a_ref[...] = jnp.dot(hn, w1a_ref[...],
                         preferred_element_type=jnp.float32) + be1_ref[...]
    b_ref[...] = jnp.dot(hn, w1b_ref[...], preferred_element_type=jnp.float32)


def _node_prep(h, w1a, w1b, be1, g1, b1):
    blk = 1000
    grid = (N // blk,)
    rowspec = pl.BlockSpec((blk, D), lambda i: (i, 0))
    wspec = pl.BlockSpec((D, H), lambda i: (0, 0))
    vspec = pl.BlockSpec((1, H), lambda i: (0, 0))
    return pl.pallas_call(
        _node_prep_body,
        grid=grid,
        in_specs=[rowspec, wspec, wspec, vspec, vspec, vspec],
        out_specs=[rowspec, rowspec, rowspec],
        out_shape=[jax.ShapeDtypeStruct((N, D), jnp.float32)] * 3,
    )(h, w1a, w1b, be1, g1, b1)


def _sc_gather(ta, tb, xp0, xp1, xp2, row2d, col2d):
    mesh = plsc.VectorSubcoreMesh(core_axis_name="core",
                                  subcore_axis_name="subcore")

    @functools.partial(
        pl.kernel,
        mesh=mesh,
        out_type=[jax.ShapeDtypeStruct((E_PAD, D), jnp.float32),
                  jax.ShapeDtypeStruct((D2R, 128), jnp.float32),
                  jax.ShapeDtypeStruct((2, HROWS, 128), jnp.float32)],
        scratch_types=[pltpu.VMEM((W_WIN // 2, D), jnp.float32),
                       pltpu.VMEM((W_WIN // 2, D), jnp.float32),
                       pltpu.VMEM((N_PAD,), jnp.float32),
                       pltpu.VMEM((N_PAD,), jnp.float32),
                       pltpu.VMEM((N_PAD,), jnp.float32),
                       pltpu.VMEM((HROWS, 128), jnp.float32),
                       pltpu.VMEM((HROWS,), jnp.int32),
                       pltpu.VMEM_SHARED((HROWS, 128), jnp.float32),
                       pltpu.SemaphoreType.DMA,
                       pltpu.SemaphoreType.DMA,
                       pltpu.SemaphoreType.DMA,
                       pltpu.SemaphoreType.DMA],
        compiler_params=_sc_compiler_params(),
    )
    def kern(ta_hbm, tb_hbm, xp0_hbm, xp1_hbm, xp2_hbm,
             row_hbm, col_hbm, u_hbm, d2_hbm, cnt_hbm,
             bbuf, bbuf2, xb0, xb1, xb2, hist, ibuf, cnt_sh,
             semA, semB, semC, semD):
        cid = lax.axis_index("core")
        sid = lax.axis_index("subcore")
        zero16 = jnp.zeros((16,), jnp.float32)
        one16 = jnp.ones((16,), jnp.float32)
        iota16 = lax.iota(jnp.int32, 16)

        pltpu.sync_copy(xp0_hbm, xb0)
        pltpu.sync_copy(xp1_hbm, xb1)
        pltpu.sync_copy(xp2_hbm, xb2)

        @pl.loop(0, HROWS)
        def _(i):
            @pl.loop(0, 128, step=16)
            def _(c):
                hist[i, pl.ds(c, 16)] = zero16

        @pl.loop(0, HROWS, step=16)
        def _(g):
            ibuf[pl.ds(g, 16)] = iota16 + g

        @pl.when(sid < 10)
        def _():
            pltpu.sync_copy(hist.at[pl.ds(0, 8)],
                            cnt_sh.at[pl.ds(sid * 8, 8)])

        plsc.subcore_barrier()

        def body(r_vmem, c_vmem, u_vmem, d2_vmem):
            hA0 = pltpu.async_copy(ta_hbm.at[r_vmem.at[0, pl.ds(0, 64)]],
                                   u_vmem.at[pl.ds(0, 64)], semA)
            hA1 = pltpu.async_copy(ta_hbm.at[r_vmem.at[0, pl.ds(64, 64)]],
                                   u_vmem.at[pl.ds(64, 64)], semD)
            hB0 = pltpu.async_copy(
                tb_hbm.at[c_vmem.at[0, pl.ds(0, 64)]], bbuf, semB)
            hB1 = pltpu.async_copy(
                tb_hbm.at[c_vmem.at[0, pl.ds(64, 64)]], bbuf2, semC)

            @pl.loop(0, W_WIN, step=16, unroll=2)
            def _(g):
                r16 = r_vmem[0, pl.ds(g, 16)]
                c16 = c_vmem[0, pl.ds(g, 16)]
                dx = (plsc.load_gather(xb0, [r16])
                      - plsc.load_gather(xb0, [c16]))
                dy = (plsc.load_gather(xb1, [r16])
                      - plsc.load_gather(xb1, [c16]))
                dz = (plsc.load_gather(xb2, [r16])
                      - plsc.load_gather(xb2, [c16]))
                d2_vmem[0, pl.ds(g, 16)] = dx * dx + dy * dy + dz * dz
                plsc.addupdate_scatter(hist, [c16 >> 7, c16 & 127], one16)

            hA0.wait()
            hB0.wait()

            @pl.loop(0, 64, unroll=4)
            def _(e):
                for c in range(0, D, 16):
                    u_vmem[e, pl.ds(c, 16)] = (u_vmem[e, pl.ds(c, 16)]
                                               + bbuf[e, pl.ds(c, 16)])

            hA1.wait()
            hB1.wait()

            @pl.loop(0, 64, unroll=4)
            def _(e):
                for c in range(0, D, 16):
                    u_vmem[64 + e, pl.ds(c, 16)] = (
                        u_vmem[64 + e, pl.ds(c, 16)]
                        + bbuf2[e, pl.ds(c, 16)])

        pltpu.emit_pipeline(
            body,
            grid=(NWIN,),
            in_specs=[pl.BlockSpec((1, W_WIN), lambda i: (0, i)),
                      pl.BlockSpec((1, W_WIN), lambda i: (0, i))],
            out_specs=[pl.BlockSpec((W_WIN, D), lambda i: (i, 0)),
                       pl.BlockSpec((1, W_WIN), lambda i: (i, 0))],
            core_axis_name=("core", "subcore"),
            dimension_semantics=(pltpu.PARALLEL,),
        )(row_hbm, col_hbm, u_hbm, d2_hbm)

        plsc.subcore_barrier()
        pltpu.sync_copy(hist, cnt_sh.at[ibuf], add=True)
        plsc.subcore_barrier()

        @pl.when(sid < 10)
        def _():
            pltpu.sync_copy(cnt_sh.at[pl.ds(sid * 8, 8)],
                            cnt_hbm.at[cid, pl.ds(sid * 8, 8)])

    return kern(ta, tb, xp0, xp1, xp2, row2d, col2d)


def _edge_mlp_body(u_ref, d2_ref, wd_ref, we2_ref, be2_ref, m_ref):
    nrow = d2_ref.shape[0]
    dist = jnp.sqrt(d2_ref[...])
    eye = (lax.broadcasted_iota(jnp.int32, (128, 128), 0)
           == lax.broadcasted_iota(jnp.int32, (128, 128), 1)
           ).astype(jnp.float32)
    dt = lax.dot_general(eye, dist, (((1,), (1,)), ((), ())),
                         preferred_element_type=jnp.float32)
    u = u_ref[...]
    wd = wd_ref[...]
    chunks = []
    for r in range(nrow):
        chunks.append(u[128 * r:128 * (r + 1), :] + dt[:, r:r + 1] * wd)
    t = _silu(jnp.concatenate(chunks, axis=0))
    m_ref[...] = _silu(jnp.dot(t, we2_ref[...],
                               preferred_element_type=jnp.float32)
                       + be2_ref[...])


def _edge_mlp(u, d2, wd, we2, be2):
    blk = 4096
    grid = (E_PAD // blk,)
    return pl.pallas_call(
        _edge_mlp_body,
        grid=grid,
        in_specs=[pl.BlockSpec((blk, D), lambda i: (i, 0)),
                  pl.BlockSpec((blk // 128, 128), lambda i: (i, 0)),
                  pl.BlockSpec((1, H), lambda i: (0, 0)),
                  pl.BlockSpec((H, H), lambda i: (0, 0)),
                  pl.BlockSpec((1, H), lambda i: (0, 0))],
        out_specs=pl.BlockSpec((blk, H), lambda i: (i, 0)),
        out_shape=jax.ShapeDtypeStruct((E_PAD, H), jnp.float32),
    )(u, d2, wd, we2, be2)


def _inv_body(c_ref, inv_ref):
    c = c_ref[0] + c_ref[1]
    inv_ref[...] = (1.0 / jnp.maximum(c, 1.0)).reshape(N_PAD)


def _inv_counts(cnt):
    return pl.pallas_call(
        _inv_body,
        in_specs=[pl.BlockSpec((2, HROWS, 128), lambda: (0, 0, 0))],
        out_specs=pl.BlockSpec((N_PAD,), lambda: (0,)),
        out_shape=jax.ShapeDtypeStruct((N_PAD,), jnp.float32),
    )(cnt)


def _sc_scatter(m, col2d, inv):
    mesh = plsc.VectorSubcoreMesh(core_axis_name="core",
                                  subcore_axis_name="subcore")

    @functools.partial(
        pl.kernel,
        mesh=mesh,
        out_type=jax.ShapeDtypeStruct((2, N_PAD, H), jnp.float32),
        scratch_types=[pltpu.VMEM_SHARED((N_PAD, H), jnp.float32),
                       pltpu.VMEM((64, H), jnp.float32),
                       pltpu.VMEM((64,), jnp.float32)],
        compiler_params=_sc_compiler_params(),
    )
    def kern(m_hbm, col_hbm, inv_hbm, sums_out, sums_sh, zbuf, ivb):
        cid = lax.axis_index("core")
        sid = lax.axis_index("subcore")
        zero16 = jnp.zeros((16,), jnp.float32)

        @pl.loop(0, 64)
        def _(i):
            @pl.loop(0, H, step=16)
            def _(c):
                zbuf[i, pl.ds(c, 16)] = zero16

        base = sid * ROWS_PER_SUB

        @pl.loop(0, ROWS_PER_SUB // 64)
        def _(j):
            pltpu.sync_copy(zbuf, sums_sh.at[pl.ds(base + j * 64, 64)])

        plsc.subcore_barrier()

        def body(m_vmem, c_vmem):
            pltpu.sync_copy(m_vmem, sums_sh.at[c_vmem.at[0]], add=True)

        pltpu.emit_pipeline(
            body,
            grid=(NWIN,),
            in_specs=[pl.BlockSpec((W_WIN, H), lambda i: (i, 0)),
                      pl.BlockSpec((1, W_WIN), lambda i: (0, i))],
            out_specs=[],
            core_axis_name=("core", "subcore"),
            dimension_semantics=(pltpu.PARALLEL,),
        )(m_hbm, col_hbm)

        plsc.subcore_barrier()

        @pl.loop(0, ROWS_PER_SUB // 64)
        def _(j):
            pltpu.sync_copy(sums_sh.at[pl.ds(base + j * 64, 64)], zbuf)
            pltpu.sync_copy(inv_hbm.at[pl.ds(base + j * 64, 64)], ivb)
            for g in range(4):
                iv16 = ivb[pl.ds(g * 16, 16)]
                for lane in range(16):
                    s = iv16[lane]
                    for c in range(0, H, 16):
                        zbuf[g * 16 + lane, pl.ds(c, 16)] = (
                            zbuf[g * 16 + lane, pl.ds(c, 16)] * s)
            pltpu.sync_copy(zbuf, sums_out.at[cid, pl.ds(base + j * 64, 64)])

    return kern(m, col2d, inv)


def _node_final_body(h_ref, hn_ref, s0_ref, s1_ref,
                     wn1h_ref, wn1m_ref, bn1_ref, wn2_ref, bn2_ref,
                     wm1_ref, bm1_ref, wm2_ref, bm2_ref, g2_ref, b2_ref,
                     out_ref):
    m_aggr = s0_ref[...] + s1_ref[...]
    hn = hn_ref[...]
    z = _silu(jnp.dot(hn, wn1h_ref[...], preferred_element_type=jnp.float32)
              + jnp.dot(m_aggr, wn1m_ref[...],
                        preferred_element_type=jnp.float32)
              + bn1_ref[...])
    h_delta = jnp.dot(z, wn2_ref[...],
                      preferred_element_type=jnp.float32) + bn2_ref[...]
    h1 = h_ref[...] + hn + h_delta
    mu = jnp.mean(h1, axis=-1, keepdims=True)
    var = jnp.mean((h1 - mu) ** 2, axis=-1, keepdims=True)
    h2n = (h1 - mu) / jnp.sqrt(var + 1e-5) * g2_ref[...] + b2_ref[...]
    h_mlp = jnp.dot(_silu(jnp.dot(h2n, wm1_ref[...],
                                  preferred_element_type=jnp.float32)
                          + bm1_ref[...]),
                    wm2_ref[...], preferred_element_type=jnp.float32) \
        + bm2_ref[...]
    out_ref[...] = h1 + h_mlp


def _node_final(h, hn, s0, s1, wn1h, wn1m, bn1, wn2, bn2,
                wm1, bm1, wm2, bm2, g2, b2):
    blk = 1000
    grid = (N // blk,)
    rowspec = pl.BlockSpec((blk, D), lambda i: (i, 0))
    wspec = pl.BlockSpec((D, H), lambda i: (0, 0))
    vspec = pl.BlockSpec((1, H), lambda i: (0, 0))
    return pl.pallas_call(
        _node_final_body,
        grid=grid,
        in_specs=[rowspec, rowspec, rowspec, rowspec,
                  wspec, wspec, vspec, wspec, vspec,
                  wspec, vspec, wspec, vspec, vspec, vspec],
        out_specs=rowspec,
        out_shape=jax.ShapeDtypeStruct((N, D), jnp.float32),
    )(h, hn, s0, s1, wn1h, wn1m, bn1, wn2, bn2,
      wm1, bm1, wm2, bm2, g2, b2)


def kernel(x, h, edge_index, W_e1, b_e1, W_e2, b_e2, W_n1, b_n1, W_n2, b_n2,
           W_m1, b_m1, W_m2, b_m2, g1, beta1, g2, beta2):
    row = edge_index[0].astype(jnp.int32)
    col = edge_index[1].astype(jnp.int32)

    w1a = W_e1[:D]
    w1b = W_e1[D:2 * D]
    wd = W_e1[2 * D]

    hn, a_tab, b_tab = _node_prep(h, w1a, w1b, b_e1.reshape(1, H),
                                  g1.reshape(1, D), beta1.reshape(1, D))

    ta = jnp.pad(a_tab, ((0, N_PAD - N), (0, 0)))
    tb = jnp.pad(b_tab, ((0, N_PAD - N), (0, 0)))
    xpad = jnp.pad(x, ((0, N_PAD - N), (0, 0)))
    xp0, xp1, xp2 = xpad[:, 0], xpad[:, 1], xpad[:, 2]

    pad = E_PAD - E
    row_p = jnp.concatenate([row, jnp.zeros((pad,), jnp.int32)]).reshape(
        1, E_PAD)
    col_p = jnp.concatenate([col, jnp.full((pad,), DUMMY, jnp.int32)]
                            ).reshape(1, E_PAD)

    u, d2, cnt_p = _sc_gather(ta, tb, xp0, xp1, xp2, row_p, col_p)

    m = _edge_mlp(u, d2, wd.reshape(1, H), W_e2, b_e2.reshape(1, H))
    inv = _inv_counts(cnt_p)

    sums_p = _sc_scatter(m, col_p, inv)

    out = _node_final(h, hn,
                      sums_p[0, :N], sums_p[1, :N],
                      W_n1[:D], W_n1[D:], b_n1.reshape(1, H),
                      W_n2, b_n2.reshape(1, D),
                      W_m1, b_m1.reshape(1, H),
                      W_m2, b_m2.reshape(1, D),
                      g2.reshape(1, D), beta2.reshape(1, D))
    return out
